# Initial kernel scaffold; baseline (speedup 1.0000x reference)
import jax
import jax.numpy as jnp
from jax.experimental import pallas as pl


def kernel(x, edge_index, batch, Wl1, Wr1, att1, b1, Wl2, Wr2, att2, b2, Wlin, blin):
    raise NotImplementedError("write your pallas kernel here")


# speedup vs baseline: 9.0227x; 9.0227x over previous
import functools

import jax
import jax.numpy as jnp
import numpy as np
from jax import lax
from jax.experimental import pallas as pl
from jax.experimental.pallas import tpu as pltpu
from jax.experimental.pallas import tpu_sc as plsc

N = 10000
C = 128
E0 = 320000
ET = E0 + N
G = 128

NC, NS, L = 2, 16, 16
NW = NC * NS
CHUNK = 128
TCH = -(-ET // (NW * CHUNK))
PT = TCH * CHUNK
EP = NW * PT
NP = 10240
STRIPE = NP // NS

_f32 = jnp.float32
_i32 = jnp.int32

_TAKE_DNUMS = jax.lax.GatherDimensionNumbers(
    offset_dims=(), collapsed_slice_dims=(0,), start_index_map=(0,))


def _lane_perm(v, p):
    return lax.gather(v, p[:, None], _TAKE_DNUMS, slice_sizes=(1,),
                      unique_indices=True,
                      mode=lax.GatherScatterMode.PROMISE_IN_BOUNDS)


def _lane_allsum(v, perms):
    for p in perms:
        v = v + _lane_perm(v, p)
    return v


_mesh = plsc.VectorSubcoreMesh(core_axis_name="c", subcore_axis_name="s")


def _sc_layer_body(xl_hbm, xr_hbm, att_hbm, src_hbm, dst_hbm,
                   num_hbm, den_hbm,
                   src_v, dst_v, wbuf, xlr, xrr, attv,
                   acc_sh, den_sh, sem_l, sem_r):
    cid = lax.axis_index("c")
    sid = lax.axis_index("s")
    wid = sid * NC + cid
    zero16 = jnp.zeros((L,), _f32)
    lane = lax.iota(_i32, L)
    bfly_perms = [lane ^ m for m in (8, 4, 2, 1)]
    lanef = lane.astype(_f32)
    one16 = zero16 + 1.0
    lane_onehot = [jnp.maximum(one16 - jnp.abs(lanef - float(j)), 0.0)
                   for j in range(L)]

    def _zrow(j, carry):
        for k in range(8):
            xlr[j, pl.ds(k * L, L)] = zero16
        return carry
    lax.fori_loop(0, CHUNK, _zrow, 0)
    for k in range(8):
        wbuf[pl.ds(k * L, L)] = zero16
    row0 = sid * STRIPE
    for b in range(STRIPE // CHUNK):
        pltpu.sync_copy(xlr, acc_sh.at[pl.ds(row0 + b * CHUNK, CHUNK)])
        pltpu.sync_copy(wbuf, den_sh.at[pl.ds(row0 + b * CHUNK, CHUNK)])
    pltpu.sync_copy(att_hbm, attv)
    plsc.subcore_barrier()

    att_regs = [attv[pl.ds(k * L, L)] for k in range(8)]

    def _chunk(ch, carry):
        ebase = wid * PT + ch * CHUNK
        pltpu.sync_copy(src_hbm.at[pl.ds(ebase, CHUNK)], src_v)
        pltpu.sync_copy(dst_hbm.at[pl.ds(ebase, CHUNK)], dst_v)
        cl = pltpu.async_copy(xl_hbm.at[src_v], xlr, sem_l)
        cr = pltpu.async_copy(xr_hbm.at[dst_v], xrr, sem_r)
        cl.wait()
        cr.wait()

        def _grp(g, gcarry):
            wl = zero16
            for j in range(L):
                e = g * L + j
                acc = None
                xs = []
                for k in range(8):
                    a = xlr[e, pl.ds(k * L, L)]
                    b_ = xrr[e, pl.ds(k * L, L)]
                    xs.append(a)
                    z = a + b_
                    z = jnp.maximum(z, 0.2 * z)
                    t = z * att_regs[k]
                    acc = t if acc is None else acc + t
                w = jnp.exp(_lane_allsum(acc, bfly_perms))
                gv = jnp.full((L,), ebase + e, _i32)
                maskf = jnp.minimum(jnp.maximum(ET - gv, 0), 1).astype(_f32)
                w = w * maskf
                wl = wl + w * lane_onehot[j]
                for k in range(8):
                    xlr[e, pl.ds(k * L, L)] = xs[k] * w
            wbuf[pl.ds(g * L, L)] = wl
            return gcarry
        lax.fori_loop(0, CHUNK // L, _grp, 0)

        pltpu.sync_copy(wbuf, den_sh.at[dst_v], add=True)
        pltpu.sync_copy(xlr, acc_sh.at[dst_v], add=True)
        return carry
    lax.fori_loop(0, TCH, _chunk, 0)

    plsc.subcore_barrier()
    pltpu.sync_copy(acc_sh.at[pl.ds(row0, STRIPE)],
                    num_hbm.at[cid, pl.ds(row0, STRIPE)])
    pltpu.sync_copy(den_sh.at[pl.ds(row0, STRIPE)],
                    den_hbm.at[cid, pl.ds(row0, STRIPE)])


_sc_layer = pl.kernel(
    _sc_layer_body,
    out_type=(jax.ShapeDtypeStruct((NC, NP, C), _f32),
              jax.ShapeDtypeStruct((NC, NP), _f32)),
    mesh=_mesh,
    scratch_types=[
        pltpu.VMEM((CHUNK,), _i32),
        pltpu.VMEM((CHUNK,), _i32),
        pltpu.VMEM((CHUNK,), _f32),
        pltpu.VMEM((CHUNK, C), _f32),
        pltpu.VMEM((CHUNK, C), _f32),
        pltpu.VMEM((C,), _f32),
        pltpu.VMEM_SHARED((NP, C), _f32),
        pltpu.VMEM_SHARED((NP,), _f32),
        pltpu.SemaphoreType.DMA,
        pltpu.SemaphoreType.DMA,
    ],
)


_RB = 1024
_GRID = NP // _RB


def _tc_pre_body(x_ref, wl_ref, wr_ref, ol_ref, or_ref):
    xb = x_ref[...]
    ol_ref[...] = jnp.dot(xb, wl_ref[...], preferred_element_type=_f32,
                     precision=lax.Precision.HIGHEST)
    or_ref[...] = jnp.dot(xb, wr_ref[...], preferred_element_type=_f32,
                     precision=lax.Precision.HIGHEST)


_tc_pre = pl.pallas_call(
    _tc_pre_body,
    grid=(_GRID,),
    in_specs=[
        pl.BlockSpec((_RB, C), lambda i: (i, 0)),
        pl.BlockSpec((C, C), lambda i: (0, 0)),
        pl.BlockSpec((C, C), lambda i: (0, 0)),
    ],
    out_specs=[
        pl.BlockSpec((_RB, C), lambda i: (i, 0)),
        pl.BlockSpec((_RB, C), lambda i: (i, 0)),
    ],
    out_shape=[jax.ShapeDtypeStruct((NP, C), _f32),
               jax.ShapeDtypeStruct((NP, C), _f32)],
)


def _tc_mid_body(num_ref, den_ref, b_ref, wl_ref, wr_ref, ol_ref, or_ref):
    n = num_ref[0] + num_ref[1]
    d = den_ref[0] + den_ref[1]
    d = jnp.maximum(d, 1e-30)
    h = jnp.maximum(n / d[:, None] + b_ref[...], 0.0)
    ol_ref[...] = jnp.dot(h, wl_ref[...], preferred_element_type=_f32,
                     precision=lax.Precision.HIGHEST)
    or_ref[...] = jnp.dot(h, wr_ref[...], preferred_element_type=_f32,
                     precision=lax.Precision.HIGHEST)


_tc_mid = pl.pallas_call(
    _tc_mid_body,
    grid=(_GRID,),
    in_specs=[
        pl.BlockSpec((NC, _RB, C), lambda i: (0, i, 0)),
        pl.BlockSpec((NC, _RB), lambda i: (0, i)),
        pl.BlockSpec((1, C), lambda i: (0, 0)),
        pl.BlockSpec((C, C), lambda i: (0, 0)),
        pl.BlockSpec((C, C), lambda i: (0, 0)),
    ],
    out_specs=[
        pl.BlockSpec((_RB, C), lambda i: (i, 0)),
        pl.BlockSpec((_RB, C), lambda i: (i, 0)),
    ],
    out_shape=[jax.ShapeDtypeStruct((NP, C), _f32),
               jax.ShapeDtypeStruct((NP, C), _f32)],
)


def _tc_post_body(num_ref, den_ref, b_ref, wlin_ref, blin_ref, batch_ref,
                  out_ref):
    i = pl.program_id(0)
    n = num_ref[0] + num_ref[1]
    d = den_ref[0] + den_ref[1]
    d = jnp.maximum(d, 1e-30)
    h = jnp.maximum(n / d[:, None] + b_ref[...], 0.0)
    y = lax.dot_general(h, wlin_ref[0], (((1,), (0,)), ((), ())),
                        preferred_element_type=_f32,
                     precision=lax.Precision.HIGHEST)
    bvec = batch_ref[0]
    onehot = (bvec[:, None] ==
              lax.broadcasted_iota(_i32, (_RB, G), 1)).astype(_f32)
    contrib = lax.dot_general(onehot, y, (((0,), (0,)), ((), ())),
                              preferred_element_type=_f32,
                     precision=lax.Precision.HIGHEST)

    @pl.when(i == 0)
    def _():
        out_ref[...] = jnp.broadcast_to(blin_ref[...], (G, G))

    out_ref[...] = out_ref[...] + contrib[:, None]


_tc_post = pl.pallas_call(
    _tc_post_body,
    grid=(_GRID,),
    in_specs=[
        pl.BlockSpec((NC, _RB, C), lambda i: (0, i, 0)),
        pl.BlockSpec((NC, _RB), lambda i: (0, i)),
        pl.BlockSpec((1, C), lambda i: (0, 0)),
        pl.BlockSpec((1, C), lambda i: (0, 0)),
        pl.BlockSpec((1, G), lambda i: (0, 0)),
        pl.BlockSpec((1, _RB), lambda i: (0, i)),
    ],
    out_specs=pl.BlockSpec((G, G), lambda i: (0, 0)),
    out_shape=jax.ShapeDtypeStruct((G, G), _f32),
)


def kernel(x, edge_index, batch, Wl1, Wr1, att1, b1, Wl2, Wr2, att2, b2,
           Wlin, blin):
    loop = jnp.arange(N, dtype=edge_index.dtype)
    padi = jnp.zeros((EP - ET,), dtype=edge_index.dtype)
    src = jnp.concatenate([edge_index[0], loop, padi])
    dst = jnp.concatenate([edge_index[1], loop, padi])
    batch_pad = jnp.concatenate(
        [batch, jnp.full((NP - N,), G, dtype=batch.dtype)]).reshape(1, NP)

    x_pad = jnp.pad(x, ((0, NP - N), (0, 0)))
    att1v = att1.reshape(C)
    att2v = att2.reshape(C)
    b1r = b1.reshape(1, C)
    b2r = b2.reshape(1, C)
    wlin_r = Wlin.reshape(1, C)
    blin_b = jnp.broadcast_to(blin.reshape(1, 1), (1, G))

    xl1, xr1 = _tc_pre(x_pad, Wl1, Wr1)
    num1, den1 = _sc_layer(xl1, xr1, att1v, src, dst)
    xl2, xr2 = _tc_mid(num1, den1, b1r, Wl2, Wr2)
    num2, den2 = _sc_layer(xl2, xr2, att2v, src, dst)
    out = _tc_post(num2, den2, b2r, wlin_r, blin_b, batch_pad)
    return out[:, :1]


# speedup vs baseline: 21.2754x; 2.3580x over previous
import functools

import jax
import jax.numpy as jnp
import numpy as np
from jax import lax
from jax.experimental import pallas as pl
from jax.experimental.pallas import tpu as pltpu
from jax.experimental.pallas import tpu_sc as plsc

N = 10000
C = 128
E0 = 320000
ET = E0 + N
G = 128

NC, NS, L = 2, 16, 16
NW = NC * NS
CHUNK = 64
TCH = 164
PT = TCH * CHUNK
EP = NW * PT
NP = 10240
NPAD_ROWS = NP - N
STRIPE = NP // NS
NGRP = CHUNK // L

_f32 = jnp.float32
_i32 = jnp.int32

_TAKE_DNUMS = jax.lax.GatherDimensionNumbers(
    offset_dims=(), collapsed_slice_dims=(0,), start_index_map=(0,))


def _lane_perm(v, p):
    return lax.gather(v, p[:, None], _TAKE_DNUMS, slice_sizes=(1,),
                      unique_indices=True,
                      mode=lax.GatherScatterMode.PROMISE_IN_BOUNDS)


def _lane_allsum(v, perms):
    for p in perms:
        v = v + _lane_perm(v, p)
    return v


_mesh = plsc.VectorSubcoreMesh(core_axis_name="c", subcore_axis_name="s")


def _sc_layer_body(xl_hbm, xr_hbm, att_hbm, src_hbm, dst_hbm,
                   num_hbm, den_hbm,
                   srcb, dstb, attv,
                   wbuf0, wbuf1, xlr0, xlr1, xrr0, xrr1,
                   acc_sh, den_sh,
                   sgl0, sgr0, sgl1, sgr1, ss0, ss1, ssd0, ssd1,
                   si0, si1, si2, si3):
    cid = lax.axis_index("c")
    sid = lax.axis_index("s")
    wid = sid * NC + cid
    zero16 = jnp.zeros((L,), _f32)
    lane = lax.iota(_i32, L)
    bfly_perms = [lane ^ m for m in (8, 4, 2, 1)]
    lanef = lane.astype(_f32)
    one16 = zero16 + 1.0
    lane_onehot = [jnp.maximum(one16 - jnp.abs(lanef - float(j)), 0.0)
                   for j in range(L)]

    xlr = (xlr0, xlr1)
    xrr = (xrr0, xrr1)
    wbuf = (wbuf0, wbuf1)
    sgl = (sgl0, sgl1)
    sgr = (sgr0, sgr1)
    ss = (ss0, ss1)
    ssd = (ssd0, ssd1)
    si = (si0, si1, si2, si3)

    def _zrow(j, carry):
        for k in range(8):
            xlr0[j, pl.ds(k * L, L)] = zero16
            xlr1[j, pl.ds(k * L, L)] = zero16
        return carry
    lax.fori_loop(0, CHUNK, _zrow, 0)
    for k in range(CHUNK // L):
        wbuf0[pl.ds(k * L, L)] = zero16
        wbuf1[pl.ds(k * L, L)] = zero16
    row0 = sid * STRIPE
    for b in range(STRIPE // CHUNK):
        pltpu.sync_copy(xlr0, acc_sh.at[pl.ds(row0 + b * CHUNK, CHUNK)])
        pltpu.sync_copy(wbuf0, den_sh.at[pl.ds(row0 + b * CHUNK, CHUNK)])
    pltpu.sync_copy(att_hbm, attv)

    att_regs = [attv[pl.ds(k * L, L)] for k in range(8)]

    def _idx_start(ch, r):
        pltpu.async_copy(src_hbm.at[wid, ch], srcb.at[r], si[r])
        pltpu.async_copy(dst_hbm.at[wid, ch], dstb.at[r], si[r])

    def _idx_wait(r):
        pltpu.make_async_copy(src_hbm.at[wid, 0], srcb.at[r], si[r]).wait()
        pltpu.make_async_copy(dst_hbm.at[wid, 0], dstb.at[r], si[r]).wait()

    def _gather_start(r, p):
        pltpu.async_copy(xl_hbm.at[srcb.at[r]], xlr[p], sgl[p])
        pltpu.async_copy(xr_hbm.at[dstb.at[r]], xrr[p], sgr[p])

    def _gather_wait(p):
        pltpu.make_async_copy(xl_hbm.at[srcb.at[0]], xlr[p], sgl[p]).wait()
        pltpu.make_async_copy(xr_hbm.at[dstb.at[0]], xrr[p], sgr[p]).wait()

    def _scatter_start(r, p):
        pltpu.sync_copy(xlr[p], acc_sh.at[dstb.at[r]], add=True)
        pltpu.sync_copy(wbuf[p], den_sh.at[dstb.at[r]], add=True)

    def _scatter_wait(p):
        pass

    def _compute(p):
        xlr_p = xlr[p]
        xrr_p = xrr[p]
        wbuf_p = wbuf[p]

        def _grp(g, gcarry):
            wl = zero16
            for j in range(L):
                e = g * L + j
                acc = None
                xs = []
                for k in range(8):
                    a = xlr_p[e, pl.ds(k * L, L)]
                    b_ = xrr_p[e, pl.ds(k * L, L)]
                    xs.append(a)
                    z = a + b_
                    z = jnp.maximum(z, 0.2 * z)
                    t = z * att_regs[k]
                    acc = t if acc is None else acc + t
                w = jnp.exp(_lane_allsum(acc, bfly_perms))
                wl = wl + w * lane_onehot[j]
                for k in range(8):
                    xlr_p[e, pl.ds(k * L, L)] = xs[k] * w
            wbuf_p[pl.ds(g * L, L)] = wl
            return gcarry
        lax.fori_loop(0, NGRP, _grp, 0)

    plsc.subcore_barrier()

    _idx_start(0, 0)
    _idx_start(1, 1)
    pltpu.sync_copy(src_hbm.at[wid, TCH], srcb.at[3])
    pltpu.sync_copy(dst_hbm.at[wid, TCH], dstb.at[3])
    _idx_wait(0)
    _gather_start(0, 0)
    _scatter_start(3, 1)

    def _outer(it, carry):
        i0 = it * 4
        for u in range(4):
            i = i0 + u
            p = u % 2
            q = 1 - p
            r0 = u
            r1 = (u + 1) % 4
            r2 = (u + 2) % 4
            _idx_start(i + 2, r2)
            _scatter_wait(q)
            _idx_wait(r1)
            _gather_start(r1, q)
            _gather_wait(p)
            _compute(p)
            _scatter_start(r0, p)
        return carry
    lax.fori_loop(0, TCH // 4, _outer, 0)

    _scatter_wait(1)
    _gather_wait(0)
    _idx_wait(1)

    plsc.subcore_barrier()
    pltpu.sync_copy(acc_sh.at[pl.ds(row0, STRIPE)],
                    num_hbm.at[cid, pl.ds(row0, STRIPE)])
    pltpu.sync_copy(den_sh.at[pl.ds(row0, STRIPE)],
                    den_hbm.at[cid, pl.ds(row0, STRIPE)])


_sc_layer = pl.kernel(
    _sc_layer_body,
    out_type=(jax.ShapeDtypeStruct((NC, NP, C), _f32),
              jax.ShapeDtypeStruct((NC, NP), _f32)),
    mesh=_mesh,
    scratch_types=[
        pltpu.VMEM((4, CHUNK), _i32),
        pltpu.VMEM((4, CHUNK), _i32),
        pltpu.VMEM((C,), _f32),
        pltpu.VMEM((CHUNK,), _f32),
        pltpu.VMEM((CHUNK,), _f32),
        pltpu.VMEM((CHUNK, C), _f32),
        pltpu.VMEM((CHUNK, C), _f32),
        pltpu.VMEM((CHUNK, C), _f32),
        pltpu.VMEM((CHUNK, C), _f32),
        pltpu.VMEM_SHARED((NP, C), _f32),
        pltpu.VMEM_SHARED((NP,), _f32),
    ] + [pltpu.SemaphoreType.DMA] * 12,
)


_RB = 1024
_GRID = NP // _RB


def _tc_pre_body(x_ref, wl_ref, wr_ref, ol_ref, or_ref):
    xb = x_ref[...]
    ol_ref[...] = jnp.dot(xb, wl_ref[...], preferred_element_type=_f32,
                     precision=lax.Precision.HIGHEST)
    or_ref[...] = jnp.dot(xb, wr_ref[...], preferred_element_type=_f32,
                     precision=lax.Precision.HIGHEST)


_tc_pre = pl.pallas_call(
    _tc_pre_body,
    grid=(_GRID,),
    in_specs=[
        pl.BlockSpec((_RB, C), lambda i: (i, 0)),
        pl.BlockSpec((C, C), lambda i: (0, 0)),
        pl.BlockSpec((C, C), lambda i: (0, 0)),
    ],
    out_specs=[
        pl.BlockSpec((_RB, C), lambda i: (i, 0)),
        pl.BlockSpec((_RB, C), lambda i: (i, 0)),
    ],
    out_shape=[jax.ShapeDtypeStruct((NP, C), _f32),
               jax.ShapeDtypeStruct((NP, C), _f32)],
)


def _tc_mid_body(num_ref, den_ref, b_ref, wl_ref, wr_ref, ol_ref, or_ref):
    n = num_ref[0] + num_ref[1]
    d = den_ref[0] + den_ref[1]
    d = jnp.maximum(d, 1e-30)
    h = jnp.maximum(n / d[:, None] + b_ref[...], 0.0)
    ol_ref[...] = jnp.dot(h, wl_ref[...], preferred_element_type=_f32,
                     precision=lax.Precision.HIGHEST)
    or_ref[...] = jnp.dot(h, wr_ref[...], preferred_element_type=_f32,
                     precision=lax.Precision.HIGHEST)


_tc_mid = pl.pallas_call(
    _tc_mid_body,
    grid=(_GRID,),
    in_specs=[
        pl.BlockSpec((NC, _RB, C), lambda i: (0, i, 0)),
        pl.BlockSpec((NC, _RB), lambda i: (0, i)),
        pl.BlockSpec((1, C), lambda i: (0, 0)),
        pl.BlockSpec((C, C), lambda i: (0, 0)),
        pl.BlockSpec((C, C), lambda i: (0, 0)),
    ],
    out_specs=[
        pl.BlockSpec((_RB, C), lambda i: (i, 0)),
        pl.BlockSpec((_RB, C), lambda i: (i, 0)),
    ],
    out_shape=[jax.ShapeDtypeStruct((NP, C), _f32),
               jax.ShapeDtypeStruct((NP, C), _f32)],
)


def _tc_post_body(num_ref, den_ref, b_ref, wlin_ref, blin_ref, batch_ref,
                  out_ref):
    i = pl.program_id(0)
    n = num_ref[0] + num_ref[1]
    d = den_ref[0] + den_ref[1]
    d = jnp.maximum(d, 1e-30)
    h = jnp.maximum(n / d[:, None] + b_ref[...], 0.0)
    y = lax.dot_general(h, wlin_ref[0], (((1,), (0,)), ((), ())),
                        preferred_element_type=_f32,
                     precision=lax.Precision.HIGHEST)
    bvec = batch_ref[0]
    onehot = (bvec[:, None] ==
              lax.broadcasted_iota(_i32, (_RB, G), 1)).astype(_f32)
    contrib = lax.dot_general(onehot, y, (((0,), (0,)), ((), ())),
                              preferred_element_type=_f32,
                     precision=lax.Precision.HIGHEST)

    @pl.when(i == 0)
    def _():
        out_ref[...] = jnp.broadcast_to(blin_ref[...], (G, G))

    out_ref[...] = out_ref[...] + contrib[:, None]


_tc_post = pl.pallas_call(
    _tc_post_body,
    grid=(_GRID,),
    in_specs=[
        pl.BlockSpec((NC, _RB, C), lambda i: (0, i, 0)),
        pl.BlockSpec((NC, _RB), lambda i: (0, i)),
        pl.BlockSpec((1, C), lambda i: (0, 0)),
        pl.BlockSpec((1, C), lambda i: (0, 0)),
        pl.BlockSpec((1, G), lambda i: (0, 0)),
        pl.BlockSpec((1, _RB), lambda i: (0, i)),
    ],
    out_specs=pl.BlockSpec((G, G), lambda i: (0, 0)),
    out_shape=jax.ShapeDtypeStruct((G, G), _f32),
)


def kernel(x, edge_index, batch, Wl1, Wr1, att1, b1, Wl2, Wr2, att2, b2,
           Wlin, blin):
    loop = jnp.arange(N, dtype=edge_index.dtype)
    padi = (N + jnp.arange(EP - ET, dtype=edge_index.dtype) % NPAD_ROWS)
    extra = (N + jnp.arange(NW * 2 * CHUNK, dtype=edge_index.dtype)
             % NPAD_ROWS).reshape(NW, 2, CHUNK)
    src = jnp.concatenate(
        [jnp.concatenate([edge_index[0], loop, padi]).reshape(NW, TCH, CHUNK),
         extra], axis=1)
    dst = jnp.concatenate(
        [jnp.concatenate([edge_index[1], loop, padi]).reshape(NW, TCH, CHUNK),
         extra], axis=1)
    batch_pad = jnp.concatenate(
        [batch, jnp.full((NP - N,), G, dtype=batch.dtype)]).reshape(1, NP)

    x_pad = jnp.pad(x, ((0, NP - N), (0, 0)))
    att1v = att1.reshape(C)
    att2v = att2.reshape(C)
    b1r = b1.reshape(1, C)
    b2r = b2.reshape(1, C)
    wlin_r = Wlin.reshape(1, C)
    blin_b = jnp.broadcast_to(blin.reshape(1, 1), (1, G))

    xl1, xr1 = _tc_pre(x_pad, Wl1, Wr1)
    num1, den1 = _sc_layer(xl1, xr1, att1v, src, dst)
    xl2, xr2 = _tc_mid(num1, den1, b1r, Wl2, Wr2)
    num2, den2 = _sc_layer(xl2, xr2, att2v, src, dst)
    out = _tc_post(num2, den2, b2r, wlin_r, blin_b, batch_pad)
    return out[:, :1]
